# Initial kernel scaffold; baseline (speedup 1.0000x reference)
import jax
import jax.numpy as jnp
from jax.experimental import pallas as pl


def kernel(h, edge_index, emb_w, emb_b, A_w, A_b, B_w, B_b, D_w, D_b, E_w, E_b, bnh_g, bnh_b, bne_g, bne_b, mlp0_w, mlp0_b, mlp1_w, mlp1_b, mlp2_w, mlp2_b):
    raise NotImplementedError("write your pallas kernel here")


# speedup vs baseline: 3.0816x; 3.0816x over previous
import functools

import jax
import jax.numpy as jnp
from jax import lax
from jax.experimental import pallas as pl
from jax.experimental.pallas import tpu as pltpu
from jax.experimental.pallas import tpu_sc as plsc

N = 10000
E = 320000
HID = 128
HALF = 64
NTILES = 16
EPAD = 327680
EPT = EPAD // NTILES
CH = 128
NCHUNK = EPT // CH
NPAD = 10112
RPT = NPAD // NTILES


def _edge_body(dtab, ebtab, srcg, dstg, dsts, zrows, out,
               acc, srcb, dstgb, dstsb, dhrows, ebrows, contrib,
               sem_a, sem_b):
  c = lax.axis_index("c")
  sid = lax.axis_index("s")

  pltpu.sync_copy(zrows.at[pl.ds(sid * RPT, RPT)],
                  acc.at[pl.ds(sid * RPT, RPT)])
  plsc.subcore_barrier()

  gbase = c * EPAD

  @pl.loop(0, NCHUNK)
  def _chunk(k):
    ebase = sid * EPT + k * CH
    pltpu.sync_copy(srcg.at[pl.ds(gbase + ebase, CH)], srcb)
    pltpu.sync_copy(dstg.at[pl.ds(gbase + ebase, CH)], dstgb)
    pltpu.sync_copy(dsts.at[pl.ds(ebase, CH)], dstsb)
    cp_eb = pltpu.async_copy(ebtab.at[srcb], ebrows, sem_a)
    cp_dh = pltpu.async_copy(dtab.at[dstgb], dhrows, sem_b)
    cp_eb.wait()
    cp_dh.wait()

    @pl.loop(0, CH)
    def _edge(e):
      for i in range(HALF // 16):
        d = dhrows[e, pl.ds(16 * i, 16)]
        eh = ebrows[e, pl.ds(16 * i, 16)]
        b = ebrows[e, pl.ds(HALF + 16 * i, 16)]
        s = 1.0 / (1.0 + jnp.exp(-(d + eh)))
        contrib[e, pl.ds(16 * i, 16)] = s * b
        contrib[e, pl.ds(HALF + 16 * i, 16)] = s

    pltpu.sync_copy(contrib, acc.at[dstsb], add=True)

  plsc.subcore_barrier()
  pltpu.sync_copy(acc.at[pl.ds(sid * RPT, RPT)],
                  out.at[pl.ds(c * NPAD + sid * RPT, RPT)])


_edge_call = pl.kernel(
    _edge_body,
    out_type=jax.ShapeDtypeStruct((2 * NPAD, HID), jnp.float32),
    mesh=plsc.VectorSubcoreMesh(core_axis_name="c", subcore_axis_name="s"),
    scratch_types=[
        pltpu.VMEM_SHARED((NPAD, HID), jnp.float32),
        pltpu.VMEM((CH,), jnp.int32),
        pltpu.VMEM((CH,), jnp.int32),
        pltpu.VMEM((CH,), jnp.int32),
        pltpu.VMEM((CH, HID), jnp.float32),
        pltpu.VMEM((CH, HID), jnp.float32),
        pltpu.VMEM((CH, HID), jnp.float32),
        pltpu.SemaphoreType.DMA,
        pltpu.SemaphoreType.DMA,
    ],
)


def _mm(x, w, b):
  return jnp.dot(x, w, preferred_element_type=jnp.float32) + b


def _write_tables(h, aw, ab, bw, bb, dw, db, ew, eb,
                  ah_out, dtab_out, ebtab_out):
  ah_out[...] = _mm(h, aw[...], ab[...])
  bh = _mm(h, bw[...], bb[...])
  dh = _mm(h, dw[...], db[...])
  ehm = _mm(h, ew[...], eb[...])
  dtab_out[0:N, :] = dh
  dtab_out[N:2 * N, :] = jnp.concatenate([dh[:, HALF:HID], dh[:, 0:HALF]],
                                         axis=1)
  ebtab_out[0:N, :] = jnp.concatenate([ehm[:, 0:HALF], bh[:, 0:HALF]], axis=1)
  ebtab_out[N:2 * N, :] = jnp.concatenate([ehm[:, HALF:HID], bh[:, HALF:HID]],
                                          axis=1)


def _tc_emb_body(h0, embw, embb, h_out):
  h_out[...] = _mm(h0[...], embw[...], embb[...])


def _tc_tables_body(h_ref, aw, ab, bw, bb, dw, db, ew, eb,
                    ah_out, dtab_out, ebtab_out):
  _write_tables(h_ref[...], aw, ab, bw, bb, dw, db, ew, eb,
                ah_out, dtab_out, ebtab_out)


def _combine_update(nd_ref, ah_ref, hin_ref, g_ref, b_ref):
  nd = nd_ref[...]
  num = jnp.concatenate([nd[0:N, 0:HALF], nd[NPAD:NPAD + N, 0:HALF]], axis=1)
  den = jnp.concatenate([nd[0:N, HALF:HID], nd[NPAD:NPAD + N, HALF:HID]],
                        axis=1)
  h = ah_ref[...] + num / (den + 1e-6)
  m = jnp.mean(h, axis=0, keepdims=True)
  v = jnp.mean((h - m) * (h - m), axis=0, keepdims=True)
  h = (h - m) / jnp.sqrt(v + 1e-5) * g_ref[...] + b_ref[...]
  return hin_ref[...] + jnp.maximum(h, 0.0)


def _tc_update_body(nd_ref, ah_ref, hin_ref, g_ref, b_ref, h_out):
  h_out[...] = _combine_update(nd_ref, ah_ref, hin_ref, g_ref, b_ref)


def _tc_last_body(nd_ref, ah_ref, hin_ref, g_ref, b_ref,
                  w0, b0, w1, b1, w2, b2, y_out):
  h = _combine_update(nd_ref, ah_ref, hin_ref, g_ref, b_ref)
  y = jnp.maximum(_mm(h, w0[...], b0[...]), 0.0)
  y = jnp.maximum(_mm(y, w1[...], b1[...]), 0.0)
  y_out[...] = _mm(y, w2[...], b2[...])


_tab_shapes = (
    jax.ShapeDtypeStruct((N, HID), jnp.float32),
    jax.ShapeDtypeStruct((2 * N, HID), jnp.float32),
    jax.ShapeDtypeStruct((2 * N, HID), jnp.float32),
)

_h_shape = jax.ShapeDtypeStruct((N, HID), jnp.float32)
_tc_emb = pl.pallas_call(_tc_emb_body, out_shape=_h_shape)
_tc_tables = pl.pallas_call(_tc_tables_body, out_shape=_tab_shapes)
_tc_update = pl.pallas_call(_tc_update_body, out_shape=_h_shape)
_tc_last = pl.pallas_call(
    _tc_last_body, out_shape=jax.ShapeDtypeStruct((N, 10), jnp.float32))


def kernel(h, edge_index, emb_w, emb_b, A_w, A_b, B_w, B_b, D_w, D_b,
           E_w, E_b, bnh_g, bnh_b, bne_g, bne_b,
           mlp0_w, mlp0_b, mlp1_w, mlp1_b, mlp2_w, mlp2_b):
  src = edge_index[0]
  dst = edge_index[1]
  npad = EPAD - E
  src_p = jnp.concatenate([src, jnp.zeros((npad,), jnp.int32)])
  dst_p = jnp.concatenate([dst, jnp.zeros((npad,), jnp.int32)])
  srcg = jnp.concatenate([src_p, src_p + N])
  dstg = jnp.concatenate([dst_p, dst_p + N])
  dsts = jnp.concatenate([dst, jnp.full((npad,), NPAD - 1, jnp.int32)])
  zrows = jnp.zeros((RPT, HID), jnp.float32)

  hh = _tc_emb(h, emb_w, emb_b)
  for l in range(4):
    ah, dtab, ebtab = _tc_tables(hh, A_w[l], A_b[l], B_w[l], B_b[l],
                                 D_w[l], D_b[l], E_w[l], E_b[l])
    nd = _edge_call(dtab, ebtab, srcg, dstg, dsts, zrows)
    if l < 3:
      hh = _tc_update(nd, ah, hh, bnh_g[l], bnh_b[l])
    else:
      y = _tc_last(nd, ah, hh, bnh_g[l], bnh_b[l],
                   mlp0_w, mlp0_b, mlp1_w, mlp1_b, mlp2_w, mlp2_b)
  return y


# speedup vs baseline: 4.3572x; 1.4139x over previous
import functools

import jax
import jax.numpy as jnp
from jax import lax
from jax.experimental import pallas as pl
from jax.experimental.pallas import tpu as pltpu
from jax.experimental.pallas import tpu_sc as plsc

N = 10000
E = 320000
HID = 128
HALF = 64
NTILES = 16
EPAD = 327680
EPT = EPAD // NTILES
CH = 80
NCHUNK = EPT // CH
NPAD = 10112
RPT = NPAD // NTILES


TCHUNKS = NTILES * NCHUNK


def _edge_body(dtab, ebtab, idxpack, out, acc,
               idxq, dh0, dh1, eb0, eb1, *sems):
  c = lax.axis_index("c")
  sid = lax.axis_index("s")

  zero = jnp.zeros((16,), jnp.float32)

  @plsc.parallel_loop(0, CH)
  def _zrow(r):
    for i in range(HID // 16):
      eb0[r, pl.ds(16 * i, 16)] = zero

  @pl.loop(0, RPT // CH)
  def _zcp(j):
    pltpu.sync_copy(eb0, acc.at[pl.ds(sid * RPT + j * CH, CH)])

  rem = RPT - (RPT // CH) * CH
  if rem:
    pltpu.sync_copy(eb0.at[pl.ds(0, rem)],
                    acc.at[pl.ds(sid * RPT + (RPT // CH) * CH, rem)])

  plsc.subcore_barrier()

  isem = sems[0:4]
  gse = sems[4:6]
  gsd = sems[6:8]
  ssem = sems[8]
  dhr = (dh0, dh1)
  ebr = (eb0, eb1)

  row0 = c * TCHUNKS + sid * NCHUNK

  def idx_start(k, q):
    pltpu.async_copy(idxpack.at[row0 + k], idxq.at[q], isem[q])

  def idx_wait(k, q):
    pltpu.make_async_copy(idxpack.at[row0 + k], idxq.at[q], isem[q]).wait()

  def gather_start(q, r):
    pltpu.async_copy(ebtab.at[idxq.at[q, 0]], ebr[r], gse[r])
    pltpu.async_copy(dtab.at[idxq.at[q, 1]], dhr[r], gsd[r])

  def gather_wait(q, r):
    pltpu.make_async_copy(ebtab.at[idxq.at[q, 0]], ebr[r], gse[r]).wait()
    pltpu.make_async_copy(dtab.at[idxq.at[q, 1]], dhr[r], gsd[r]).wait()

  def scatter_start(q, r):
    pltpu.async_copy(ebr[r], acc.at[idxq.at[q, 2]], ssem, add=True)

  def scatter_wait(q, r):
    pltpu.make_async_copy(ebr[r], acc.at[idxq.at[q, 2]], ssem).wait()

  idx_start(0, 0)
  idx_wait(0, 0)
  gather_start(0, 0)
  idx_start(1, 1)

  @pl.loop(0, NCHUNK, step=4)
  def _outer(k0):
    for b in range(4):
      k = k0 + b
      r = b % 2
      buf_dh = dhr[r]
      buf_eb = ebr[r]

      gather_wait(b, r)

      @pl.when(k >= 1)
      def _():
        scatter_wait((b + 3) % 4, 1 - r)

      @pl.when(k + 1 < NCHUNK)
      def _():
        idx_wait(k + 1, (b + 1) % 4)
        gather_start((b + 1) % 4, 1 - r)

      @pl.when(k + 2 < NCHUNK)
      def _():
        idx_start(k + 2, (b + 2) % 4)

      @plsc.parallel_loop(0, CH, unroll=2)
      def _edge(e):
        for i in range(HALF // 16):
          d = buf_dh[e, pl.ds(16 * i, 16)]
          eh = buf_eb[e, pl.ds(16 * i, 16)]
          bv = buf_eb[e, pl.ds(HALF + 16 * i, 16)]
          s = 1.0 / (1.0 + jnp.exp(-(d + eh)))
          buf_eb[e, pl.ds(16 * i, 16)] = s * bv
          buf_eb[e, pl.ds(HALF + 16 * i, 16)] = s

      scatter_start(b, r)

  scatter_wait(3, 1)

  plsc.subcore_barrier()
  pltpu.sync_copy(acc.at[pl.ds(sid * RPT, RPT)],
                  out.at[pl.ds(c * NPAD + sid * RPT, RPT)])


_edge_call = pl.kernel(
    _edge_body,
    out_type=jax.ShapeDtypeStruct((2 * NPAD, HID), jnp.float32),
    mesh=plsc.VectorSubcoreMesh(core_axis_name="c", subcore_axis_name="s"),
    scratch_types=[
        pltpu.VMEM_SHARED((NPAD, HID), jnp.float32),
        pltpu.VMEM((4, 3, CH), jnp.int32),
        pltpu.VMEM((CH, HID), jnp.float32),
        pltpu.VMEM((CH, HID), jnp.float32),
        pltpu.VMEM((CH, HID), jnp.float32),
        pltpu.VMEM((CH, HID), jnp.float32),
    ] + [pltpu.SemaphoreType.DMA] * 9,
)


def _mm(x, w, b):
  return jnp.dot(x, w, preferred_element_type=jnp.float32) + b


def _write_tables(h, aw, ab, bw, bb, dw, db, ew, eb,
                  ah_out, dtab_out, ebtab_out):
  ah_out[...] = _mm(h, aw[...], ab[...])
  bh = _mm(h, bw[...], bb[...])
  dh = _mm(h, dw[...], db[...])
  ehm = _mm(h, ew[...], eb[...])
  dtab_out[0:N, :] = dh
  dtab_out[N:2 * N, :] = jnp.concatenate([dh[:, HALF:HID], dh[:, 0:HALF]],
                                         axis=1)
  ebtab_out[0:N, :] = jnp.concatenate([ehm[:, 0:HALF], bh[:, 0:HALF]], axis=1)
  ebtab_out[N:2 * N, :] = jnp.concatenate([ehm[:, HALF:HID], bh[:, HALF:HID]],
                                          axis=1)


def _tc_emb_body(h0, embw, embb, h_out):
  h_out[...] = _mm(h0[...], embw[...], embb[...])


def _tc_tables_body(h_ref, aw, ab, bw, bb, dw, db, ew, eb,
                    ah_out, dtab_out, ebtab_out):
  _write_tables(h_ref[...], aw, ab, bw, bb, dw, db, ew, eb,
                ah_out, dtab_out, ebtab_out)


def _combine_update(nd_ref, ah_ref, hin_ref, g_ref, b_ref):
  nd = nd_ref[...]
  num = jnp.concatenate([nd[0:N, 0:HALF], nd[NPAD:NPAD + N, 0:HALF]], axis=1)
  den = jnp.concatenate([nd[0:N, HALF:HID], nd[NPAD:NPAD + N, HALF:HID]],
                        axis=1)
  h = ah_ref[...] + num / (den + 1e-6)
  m = jnp.mean(h, axis=0, keepdims=True)
  v = jnp.mean((h - m) * (h - m), axis=0, keepdims=True)
  h = (h - m) / jnp.sqrt(v + 1e-5) * g_ref[...] + b_ref[...]
  return hin_ref[...] + jnp.maximum(h, 0.0)


def _tc_update_body(nd_ref, ah_ref, hin_ref, g_ref, b_ref, h_out):
  h_out[...] = _combine_update(nd_ref, ah_ref, hin_ref, g_ref, b_ref)


def _tc_last_body(nd_ref, ah_ref, hin_ref, g_ref, b_ref,
                  w0, b0, w1, b1, w2, b2, y_out):
  h = _combine_update(nd_ref, ah_ref, hin_ref, g_ref, b_ref)
  y = jnp.maximum(_mm(h, w0[...], b0[...]), 0.0)
  y = jnp.maximum(_mm(y, w1[...], b1[...]), 0.0)
  y_out[...] = _mm(y, w2[...], b2[...])


_tab_shapes = (
    jax.ShapeDtypeStruct((N, HID), jnp.float32),
    jax.ShapeDtypeStruct((2 * N, HID), jnp.float32),
    jax.ShapeDtypeStruct((2 * N, HID), jnp.float32),
)

_h_shape = jax.ShapeDtypeStruct((N, HID), jnp.float32)
_tc_emb = pl.pallas_call(_tc_emb_body, out_shape=_h_shape)
_tc_tables = pl.pallas_call(_tc_tables_body, out_shape=_tab_shapes)
_tc_update = pl.pallas_call(_tc_update_body, out_shape=_h_shape)
_tc_last = pl.pallas_call(
    _tc_last_body, out_shape=jax.ShapeDtypeStruct((N, 10), jnp.float32))


def kernel(h, edge_index, emb_w, emb_b, A_w, A_b, B_w, B_b, D_w, D_b,
           E_w, E_b, bnh_g, bnh_b, bne_g, bne_b,
           mlp0_w, mlp0_b, mlp1_w, mlp1_b, mlp2_w, mlp2_b):
  src = edge_index[0]
  dst = edge_index[1]
  npad = EPAD - E
  src_p = jnp.concatenate([src, jnp.zeros((npad,), jnp.int32)])
  dst_p = jnp.concatenate([dst, jnp.zeros((npad,), jnp.int32)])
  dsts = jnp.concatenate([dst, jnp.full((npad,), NPAD - 1, jnp.int32)])
  coff = jnp.array([[0], [N]], jnp.int32)
  sg = (src_p[None, :] + coff).reshape(2, TCHUNKS, CH)
  dg = (dst_p[None, :] + coff).reshape(2, TCHUNKS, CH)
  ds2 = jnp.broadcast_to(dsts[None, :], (2, EPAD)).reshape(2, TCHUNKS, CH)
  idxpack = jnp.stack([sg, dg, ds2], axis=2).reshape(2 * TCHUNKS, 3, CH)

  hh = _tc_emb(h, emb_w, emb_b)
  for l in range(4):
    ah, dtab, ebtab = _tc_tables(hh, A_w[l], A_b[l], B_w[l], B_b[l],
                                 D_w[l], D_b[l], E_w[l], E_b[l])
    nd = _edge_call(dtab, ebtab, idxpack)
    if l < 3:
      hh = _tc_update(nd, ah, hh, bnh_g[l], bnh_b[l])
    else:
      y = _tc_last(nd, ah, hh, bnh_g[l], bnh_b[l],
                   mlp0_w, mlp0_b, mlp1_w, mlp1_b, mlp2_w, mlp2_b)
  return y
